# baseline (speedup 1.0000x reference)
import functools

import jax
import jax.numpy as jnp
from jax import lax
from jax.experimental import pallas as pl
from jax.experimental.pallas import tpu as pltpu
from jax.experimental.pallas import tpu_sc as plsc

MASK_RATIO_ = 0.75
R, C = 64, 128
KTILE = 2048
SC_CHUNK = 16


def _xor_shuffle(x, d):
    if d < C:
        bit = jax.lax.broadcasted_iota(jnp.int32, (R, C), 1) & d
        return jnp.where(bit != 0, pltpu.roll(x, d, 1), pltpu.roll(x, C - d, 1))
    s = d // C
    bit = jax.lax.broadcasted_iota(jnp.int32, (R, C), 0) & s
    return jnp.where(bit != 0, pltpu.roll(x, s, 0), pltpu.roll(x, R - s, 0))


def _sort_kernel(noise_ref, shuf_ref, *, L):
    ir = jax.lax.broadcasted_iota(jnp.int32, (R, C), 0)
    ic = jax.lax.broadcasted_iota(jnp.int32, (R, C), 1)
    idx = ir * C + ic
    m = (noise_ref[0, :, :] * float(1 << 23)).astype(jnp.int32)

    def bit_of(v):
        if v < C:
            return (ic & v) != 0
        return (ir & (v // C)) != 0

    k = 2
    while k <= L:
        d = k // 2
        while d >= 1:
            pm = _xor_shuffle(m, d)
            pidx = _xor_shuffle(idx, d)
            p_lt = (pm < m) | ((pm == m) & (pidx < idx))
            want_min = bit_of(k) == bit_of(d)
            take = p_lt == want_min
            m = jnp.where(take, pm, m)
            idx = jnp.where(take, pidx, idx)
            d //= 2
        k *= 2
    shuf_ref[0, :, :] = idx


def _restore_kernel(shrow_ref, shcol_ref, rest_ref, mask_ref, *, L, len_keep):
    sh_row = shrow_ref[0, :, :]
    sh_col = shcol_ref[0, :, :]
    ihi = jax.lax.broadcasted_iota(jnp.int32, (R, 1), 0)
    ilo = jax.lax.broadcasted_iota(jnp.int32, (1, C), 1)
    a = ((sh_row >> 7) == ihi).astype(jnp.float32)
    kcol = jax.lax.broadcasted_iota(jnp.int32, (L, 1), 0)
    b = jnp.where((sh_col & (C - 1)) == ilo, kcol, 0).astype(jnp.float32)
    mres = jnp.dot(a, b, precision=jax.lax.Precision.HIGHEST)
    rest = mres.astype(jnp.int32)
    rest_ref[0, :, :] = rest
    mask_ref[0, :, :] = jnp.where(mres >= float(len_keep), 1.0, 0.0)


def _bcast_kernel(col_ref, out_ref, *, E, rows):
    out_ref[0, :, :] = jnp.broadcast_to(col_ref[0, :, :], (rows, E))


def _lane_splat(v16, lane):
    idx = jnp.full((16, 1), lane, dtype=jnp.int32)
    return lax.gather(
        v16, idx,
        dimension_numbers=lax.GatherDimensionNumbers(
            offset_dims=(), collapsed_slice_dims=(0,), start_index_map=(0,)),
        slice_sizes=(1,),
        mode=lax.GatherScatterMode.PROMISE_IN_BOUNDS)


def _make_sc_writer(nrows, E):
    info = plsc.get_sparse_core_info()
    NC, NS = info.num_cores, info.num_subcores
    NW = NC * NS
    rows_w = nrows // NW
    nch = rows_w // SC_CHUNK
    assert rows_w % SC_CHUNK == 0 and nch % 2 == 0 and SC_CHUNK == 16
    groups = E // 16
    mesh = plsc.VectorSubcoreMesh(core_axis_name="c", subcore_axis_name="s")

    @functools.partial(
        pl.kernel, mesh=mesh,
        out_type=jax.ShapeDtypeStruct((nrows * E,), jnp.int32),
        scratch_types=[
            pltpu.VMEM((rows_w,), jnp.int32),
            pltpu.VMEM((SC_CHUNK * E,), jnp.int32),
            pltpu.VMEM((SC_CHUNK * E,), jnp.int32),
            pltpu.SemaphoreType.DMA,
            pltpu.SemaphoreType.DMA,
        ],
    )
    def writer(vals_hbm, out_hbm, vals_v, buf0, buf1, sem0, sem1):
        wid = lax.axis_index("s") * NC + lax.axis_index("c")
        base = wid * rows_w
        pltpu.sync_copy(vals_hbm.at[pl.ds(base, rows_w)], vals_v)
        bufs, sems = (buf0, buf1), (sem0, sem1)

        def fill_fire(c, b2):
            buf = bufs[b2]
            v16 = vals_v[pl.ds(c * SC_CHUNK, SC_CHUNK)]
            for r in range(SC_CHUNK):
                v = _lane_splat(v16, r)
                for j in range(groups):
                    buf[pl.ds(r * E + j * 16, 16)] = v
            pltpu.make_async_copy(
                buf,
                out_hbm.at[pl.ds((base + c * SC_CHUNK) * E, SC_CHUNK * E)],
                sems[b2]).start()

        def wait_b(b2):
            pltpu.make_async_copy(
                bufs[b2],
                out_hbm.at[pl.ds(base * E, SC_CHUNK * E)],
                sems[b2]).wait()

        fill_fire(0, 0)
        fill_fire(1, 1)

        def body(t, _):
            for b2 in range(2):
                c = t * 2 + b2
                wait_b(b2)
                fill_fire(c, b2)
            return _

        lax.fori_loop(1, nch // 2, body, None)
        wait_b(0)
        wait_b(1)

    return writer


def kernel(x):
    B, L, E = x.shape
    len_keep = int(L * (1.0 - MASK_RATIO_))
    noise = jax.random.uniform(
        jax.random.fold_in(jax.random.key(0), 1), (B, L), dtype=jnp.float32)
    noise_g = noise.reshape(B, R, C)

    shuf = pl.pallas_call(
        functools.partial(_sort_kernel, L=L),
        grid=(B,),
        in_specs=[pl.BlockSpec((1, R, C), lambda b: (b, 0, 0))],
        out_specs=pl.BlockSpec((1, R, C), lambda b: (b, 0, 0)),
        out_shape=jax.ShapeDtypeStruct((B, R, C), jnp.int32),
    )(noise_g)

    sh_row = shuf.reshape(B, 1, L)
    sh_col = shuf.reshape(B, L, 1)

    rest, mask = pl.pallas_call(
        functools.partial(_restore_kernel, L=L, len_keep=len_keep),
        grid=(B,),
        in_specs=[
            pl.BlockSpec((1, 1, L), lambda b: (b, 0, 0)),
            pl.BlockSpec((1, L, 1), lambda b: (b, 0, 0)),
        ],
        out_specs=[
            pl.BlockSpec((1, R, C), lambda b: (b, 0, 0)),
            pl.BlockSpec((1, R, C), lambda b: (b, 0, 0)),
        ],
        out_shape=[
            jax.ShapeDtypeStruct((B, R, C), jnp.int32),
            jax.ShapeDtypeStruct((B, R, C), jnp.float32),
        ],
    )(sh_row, sh_col)

    def bcast(nrows, off):
        rows = min(KTILE, nrows)
        return pl.pallas_call(
            functools.partial(_bcast_kernel, E=E, rows=rows),
            grid=(B, nrows // rows),
            in_specs=[pl.BlockSpec((1, rows, 1), lambda b, k: (b, k + off, 0))],
            out_specs=pl.BlockSpec((1, rows, E), lambda b, k: (b, k, 0)),
            out_shape=jax.ShapeDtypeStruct((B, nrows, E), jnp.int32),
        )(sh_col)

    mask_vals = shuf.reshape(B, L)[:, len_keep:].reshape(B * (L - len_keep))
    ids_mask = _make_sc_writer(B * (L - len_keep), E)(
        mask_vals).reshape(B, L - len_keep, E)
    ids_keep = bcast(len_keep, 0)

    return (mask.reshape(B, L), ids_keep, rest.reshape(B, L), ids_mask)


# speedup vs baseline: 1.9050x; 1.9050x over previous
import functools

import jax
import jax.numpy as jnp
from jax.experimental import pallas as pl
from jax.experimental.pallas import tpu as pltpu

MASK_RATIO_ = 0.75
R, C = 64, 128
KTILE = 2048


def _xor_shuffle(x, d):
    if d < C:
        bit = jax.lax.broadcasted_iota(jnp.int32, (R, C), 1) & d
        return jnp.where(bit != 0, pltpu.roll(x, d, 1), pltpu.roll(x, C - d, 1))
    s = d // C
    bit = jax.lax.broadcasted_iota(jnp.int32, (R, C), 0) & s
    return jnp.where(bit != 0, pltpu.roll(x, s, 0), pltpu.roll(x, R - s, 0))


def _sort_kernel(noise_ref, shuf_ref, *, L):
    ir = jax.lax.broadcasted_iota(jnp.int32, (R, C), 0)
    ic = jax.lax.broadcasted_iota(jnp.int32, (R, C), 1)
    idx = ir * C + ic
    m = (noise_ref[0, :, :] * float(1 << 23)).astype(jnp.int32)

    def bit_of(v):
        if v < C:
            return (ic & v) != 0
        return (ir & (v // C)) != 0

    k = 2
    while k <= L:
        d = k // 2
        while d >= 1:
            pm = _xor_shuffle(m, d)
            pidx = _xor_shuffle(idx, d)
            p_lt = (pm < m) | ((pm == m) & (pidx < idx))
            want_min = bit_of(k) == bit_of(d)
            take = p_lt == want_min
            m = jnp.where(take, pm, m)
            idx = jnp.where(take, pidx, idx)
            d //= 2
        k *= 2
    shuf_ref[0, :, :] = idx


def _restore_kernel(shrow_ref, shcol_ref, rest_ref, mask_ref, *, L, len_keep):
    sh_row = shrow_ref[0, :, :]
    sh_col = shcol_ref[0, :, :]
    ihi = jax.lax.broadcasted_iota(jnp.int32, (R, 1), 0)
    ilo = jax.lax.broadcasted_iota(jnp.int32, (1, C), 1)
    a = ((sh_row >> 7) == ihi).astype(jnp.float32)
    kcol = jax.lax.broadcasted_iota(jnp.int32, (L, 1), 0)
    b = jnp.where((sh_col & (C - 1)) == ilo, kcol, 0).astype(jnp.float32)
    mres = jnp.dot(a, b, precision=jax.lax.Precision.HIGHEST)
    rest = mres.astype(jnp.int32)
    rest_ref[0, :, :] = rest
    mask_ref[0, :, :] = jnp.where(mres >= float(len_keep), 1.0, 0.0)


def _bcast_kernel(col_ref, out_ref, *, E, rows):
    out_ref[0, :, :] = jnp.broadcast_to(col_ref[0, :, :], (rows, E))


def _bcast2_kernel(col_ref, keep_ref, maskk_ref, *, E, krows, mrows, len_keep):
    t = pl.program_id(1)
    kq = col_ref[0, pl.ds(t * krows, krows), :]
    mq = col_ref[0, pl.ds(len_keep + t * mrows, mrows), :]
    keep_ref[0, :, :] = jnp.broadcast_to(kq, (krows, E))
    maskk_ref[0, :, :] = jnp.broadcast_to(mq, (mrows, E))


def kernel(x):
    B, L, E = x.shape
    len_keep = int(L * (1.0 - MASK_RATIO_))
    noise = jax.random.uniform(
        jax.random.fold_in(jax.random.key(0), 1), (B, L), dtype=jnp.float32)
    noise_g = noise.reshape(B, R, C)

    shuf = pl.pallas_call(
        functools.partial(_sort_kernel, L=L),
        grid=(B,),
        in_specs=[pl.BlockSpec((1, R, C), lambda b: (b, 0, 0))],
        out_specs=pl.BlockSpec((1, R, C), lambda b: (b, 0, 0)),
        out_shape=jax.ShapeDtypeStruct((B, R, C), jnp.int32),
    )(noise_g)

    sh_row = shuf.reshape(B, 1, L)
    sh_col = shuf.reshape(B, L, 1)

    rest, mask = pl.pallas_call(
        functools.partial(_restore_kernel, L=L, len_keep=len_keep),
        grid=(B,),
        in_specs=[
            pl.BlockSpec((1, 1, L), lambda b: (b, 0, 0)),
            pl.BlockSpec((1, L, 1), lambda b: (b, 0, 0)),
        ],
        out_specs=[
            pl.BlockSpec((1, R, C), lambda b: (b, 0, 0)),
            pl.BlockSpec((1, R, C), lambda b: (b, 0, 0)),
        ],
        out_shape=[
            jax.ShapeDtypeStruct((B, R, C), jnp.int32),
            jax.ShapeDtypeStruct((B, R, C), jnp.float32),
        ],
    )(sh_row, sh_col)

    def bcast(nrows, off):
        rows = min(KTILE, nrows)
        return pl.pallas_call(
            functools.partial(_bcast_kernel, E=E, rows=rows),
            grid=(B, nrows // rows),
            in_specs=[pl.BlockSpec((1, rows, 1), lambda b, k: (b, k + off, 0))],
            out_specs=pl.BlockSpec((1, rows, E), lambda b, k: (b, k, 0)),
            out_shape=jax.ShapeDtypeStruct((B, nrows, E), jnp.int32),
        )(sh_col)

    del bcast
    NT = 4
    krows, mrows = len_keep // NT, (L - len_keep) // NT
    ids_keep, ids_mask = pl.pallas_call(
        functools.partial(_bcast2_kernel, E=E, krows=krows, mrows=mrows,
                          len_keep=len_keep),
        grid=(B, NT),
        in_specs=[pl.BlockSpec((1, L, 1), lambda b, t: (b, 0, 0))],
        out_specs=[
            pl.BlockSpec((1, krows, E), lambda b, t: (b, t, 0)),
            pl.BlockSpec((1, mrows, E), lambda b, t: (b, t, 0)),
        ],
        out_shape=[
            jax.ShapeDtypeStruct((B, len_keep, E), jnp.int32),
            jax.ShapeDtypeStruct((B, L - len_keep, E), jnp.int32),
        ],
    )(sh_col)

    return (mask.reshape(B, L), ids_keep, rest.reshape(B, L), ids_mask)


# speedup vs baseline: 2.4091x; 1.2646x over previous
import functools

import jax
import jax.numpy as jnp
from jax.experimental import pallas as pl
from jax.experimental.pallas import tpu as pltpu

MASK_RATIO_ = 0.75
R, C = 64, 128
KTILE = 2048


def _xor_shuffle(x, d):
    if d < C:
        bit = jax.lax.broadcasted_iota(jnp.int32, (R, C), 1) & d
        return jnp.where(bit != 0, pltpu.roll(x, d, 1), pltpu.roll(x, C - d, 1))
    s = d // C
    bit = jax.lax.broadcasted_iota(jnp.int32, (R, C), 0) & s
    return jnp.where(bit != 0, pltpu.roll(x, s, 0), pltpu.roll(x, R - s, 0))


def _sort_kernel(noise_ref, shuf_ref, *, L):
    ir = jax.lax.broadcasted_iota(jnp.int32, (R, C), 0)
    ic = jax.lax.broadcasted_iota(jnp.int32, (R, C), 1)
    idx = ir * C + ic
    m = (noise_ref[0, :, :] * float(1 << 23)).astype(jnp.int32)

    def bit_of(v):
        if v < C:
            return (ic & v) != 0
        return (ir & (v // C)) != 0

    k = 2
    while k <= L:
        d = k // 2
        while d >= 1:
            pm = _xor_shuffle(m, d)
            pidx = _xor_shuffle(idx, d)
            p_lt = (pm < m) | ((pm == m) & (pidx < idx))
            want_min = bit_of(k) == bit_of(d)
            take = p_lt == want_min
            m = jnp.where(take, pm, m)
            idx = jnp.where(take, pidx, idx)
            d //= 2
        k *= 2
    shuf_ref[0, :, :] = idx


def _mega_kernel(shg_ref, rest_ref, mask_ref, keep_ref, maskk_ref, colscr,
                 *, L, E, len_keep, krows, mrows):
    t = pl.program_id(1)

    @pl.when(t == 0)
    def _():
        sh = shg_ref[0, :, :]
        ident = (jax.lax.broadcasted_iota(jnp.int32, (C, C), 0) ==
                 jax.lax.broadcasted_iota(jnp.int32, (C, C), 1)
                 ).astype(jnp.float32)
        colmat = jax.lax.dot_general(
            ident, sh.astype(jnp.float32), (((1,), (1,)), ((), ())),
            precision=jax.lax.Precision.HIGHEST)
        colmat_i = colmat.astype(jnp.int32)
        for r in range(R):
            colscr[pl.ds(r * C, C), :] = colmat_i[:, r:r + 1]
        col = colscr[:, :]
        sh_row = jnp.concatenate(
            [sh[r:r + 1, :] for r in range(R)], axis=1)
        ihi = jax.lax.broadcasted_iota(jnp.int32, (R, 1), 0)
        ilo = jax.lax.broadcasted_iota(jnp.int32, (1, C), 1)
        a = ((sh_row >> 7) == ihi).astype(jnp.float32)
        kcol = jax.lax.broadcasted_iota(jnp.int32, (L, 1), 0)
        b = jnp.where((col & (C - 1)) == ilo, kcol, 0).astype(jnp.float32)
        mres = jnp.dot(a, b, precision=jax.lax.Precision.HIGHEST)
        rest_ref[0, :, :] = mres.astype(jnp.int32)
        mask_ref[0, :, :] = jnp.where(mres >= float(len_keep), 1.0, 0.0)

    kq = colscr[pl.ds(t * krows, krows), :]
    mq = colscr[pl.ds(len_keep + t * mrows, mrows), :]
    keep_ref[0, :, :] = jnp.broadcast_to(kq, (krows, E))
    maskk_ref[0, :, :] = jnp.broadcast_to(mq, (mrows, E))


def kernel(x):
    B, L, E = x.shape
    len_keep = int(L * (1.0 - MASK_RATIO_))
    noise = jax.random.uniform(
        jax.random.fold_in(jax.random.key(0), 1), (B, L), dtype=jnp.float32)
    noise_g = noise.reshape(B, R, C)

    shuf = pl.pallas_call(
        functools.partial(_sort_kernel, L=L),
        grid=(B,),
        in_specs=[pl.BlockSpec((1, R, C), lambda b: (b, 0, 0))],
        out_specs=pl.BlockSpec((1, R, C), lambda b: (b, 0, 0)),
        out_shape=jax.ShapeDtypeStruct((B, R, C), jnp.int32),
    )(noise_g)

    NT = 4
    krows, mrows = len_keep // NT, (L - len_keep) // NT
    rest, mask, ids_keep, ids_mask = pl.pallas_call(
        functools.partial(_mega_kernel, L=L, E=E, len_keep=len_keep,
                          krows=krows, mrows=mrows),
        grid=(B, NT),
        in_specs=[pl.BlockSpec((1, R, C), lambda b, t: (b, 0, 0))],
        out_specs=[
            pl.BlockSpec((1, R, C), lambda b, t: (b, 0, 0)),
            pl.BlockSpec((1, R, C), lambda b, t: (b, 0, 0)),
            pl.BlockSpec((1, krows, E), lambda b, t: (b, t, 0)),
            pl.BlockSpec((1, mrows, E), lambda b, t: (b, t, 0)),
        ],
        out_shape=[
            jax.ShapeDtypeStruct((B, R, C), jnp.int32),
            jax.ShapeDtypeStruct((B, R, C), jnp.float32),
            jax.ShapeDtypeStruct((B, len_keep, E), jnp.int32),
            jax.ShapeDtypeStruct((B, L - len_keep, E), jnp.int32),
        ],
        scratch_shapes=[pltpu.VMEM((L, 1), jnp.int32)],
    )(shuf)

    return (mask.reshape(B, L), ids_keep, rest.reshape(B, L), ids_mask)


# speedup vs baseline: 2.5122x; 1.0428x over previous
import functools

import jax
import jax.numpy as jnp
from jax.experimental import pallas as pl
from jax.experimental.pallas import tpu as pltpu

MASK_RATIO_ = 0.75
R, C = 64, 128
KTILE = 2048


def _xor_shuffle(x, d):
    if d < C:
        bit = jax.lax.broadcasted_iota(jnp.int32, (R, C), 1) & d
        return jnp.where(bit != 0, pltpu.roll(x, d, 1), pltpu.roll(x, C - d, 1))
    s = d // C
    bit = jax.lax.broadcasted_iota(jnp.int32, (R, C), 0) & s
    return jnp.where(bit != 0, pltpu.roll(x, s, 0), pltpu.roll(x, R - s, 0))


def _sort_kernel(noise_ref, shuf_ref, *, L):
    ir = jax.lax.broadcasted_iota(jnp.int32, (R, C), 0)
    ic = jax.lax.broadcasted_iota(jnp.int32, (R, C), 1)
    idx = ir * C + ic
    m = (noise_ref[0, :, :] * float(1 << 23)).astype(jnp.int32)

    def bit_of(v):
        if v < C:
            return (ic & v) != 0
        return (ir & (v // C)) != 0

    k = 2
    while k <= L:
        d = k // 2
        while d >= 1:
            pm = _xor_shuffle(m, d)
            pidx = _xor_shuffle(idx, d)
            want_min = bit_of(k) == bit_of(d)
            take = (want_min & (pm < m)) | (~want_min & (m < pm))
            m = jnp.where(take, pm, m)
            idx = jnp.where(take, pidx, idx)
            d //= 2
        k *= 2

    ic1 = (ic & 1) != 0

    def nxt(x):
        a = pltpu.roll(x, C - 1, 1)
        return jnp.where(ic == C - 1, pltpu.roll(a, R - 1, 0), a)

    def prv(x):
        p = pltpu.roll(x, 1, 1)
        return jnp.where(ic == 0, pltpu.roll(p, 1, 0), p)

    mnext = nxt(m)
    mprev = prv(m)
    for parity in (0, 1, 0):
        inext = nxt(idx)
        iprev = prv(idx)
        is_low = ic1 if parity else jnp.logical_not(ic1)
        mp = jnp.where(is_low, mnext, mprev)
        idxp = jnp.where(is_low, inext, iprev)
        fixed = jnp.where(is_low, jnp.minimum(idx, idxp),
                          jnp.maximum(idx, idxp))
        idx = jnp.where(mp == m, fixed, idx)
    shuf_ref[0, :, :] = idx


def _mega_kernel(shg_ref, rest_ref, mask_ref, keep_ref, maskk_ref, colscr,
                 *, L, E, len_keep, krows, mrows):
    t = pl.program_id(1)

    @pl.when(t == 0)
    def _():
        sh = shg_ref[0, :, :]
        ident = (jax.lax.broadcasted_iota(jnp.int32, (C, C), 0) ==
                 jax.lax.broadcasted_iota(jnp.int32, (C, C), 1)
                 ).astype(jnp.float32)
        colmat = jax.lax.dot_general(
            ident, sh.astype(jnp.float32), (((1,), (1,)), ((), ())),
            precision=jax.lax.Precision.HIGHEST)
        colmat_i = colmat.astype(jnp.int32)
        for r in range(R):
            colscr[pl.ds(r * C, C), :] = colmat_i[:, r:r + 1]
        col = colscr[:, :]
        sh_row = jnp.concatenate(
            [sh[r:r + 1, :] for r in range(R)], axis=1)
        ihi = jax.lax.broadcasted_iota(jnp.int32, (R, 1), 0)
        ilo = jax.lax.broadcasted_iota(jnp.int32, (1, C), 1)
        a = ((sh_row >> 7) == ihi).astype(jnp.float32)
        kcol = jax.lax.broadcasted_iota(jnp.int32, (L, 1), 0)
        b = jnp.where((col & (C - 1)) == ilo, kcol, 0).astype(jnp.float32)
        mres = jnp.dot(a, b, precision=jax.lax.Precision.HIGHEST)
        rest_ref[0, :, :] = mres.astype(jnp.int32)
        mask_ref[0, :, :] = jnp.where(mres >= float(len_keep), 1.0, 0.0)

    kq = colscr[pl.ds(t * krows, krows), :]
    mq = colscr[pl.ds(len_keep + t * mrows, mrows), :]
    keep_ref[0, :, :] = jnp.broadcast_to(kq, (krows, E))
    maskk_ref[0, :, :] = jnp.broadcast_to(mq, (mrows, E))


def kernel(x):
    B, L, E = x.shape
    len_keep = int(L * (1.0 - MASK_RATIO_))
    noise = jax.random.uniform(
        jax.random.fold_in(jax.random.key(0), 1), (B, L), dtype=jnp.float32)
    noise_g = noise.reshape(B, R, C)

    shuf = pl.pallas_call(
        functools.partial(_sort_kernel, L=L),
        grid=(B,),
        in_specs=[pl.BlockSpec((1, R, C), lambda b: (b, 0, 0))],
        out_specs=pl.BlockSpec((1, R, C), lambda b: (b, 0, 0)),
        out_shape=jax.ShapeDtypeStruct((B, R, C), jnp.int32),
    )(noise_g)

    NT = 4
    krows, mrows = len_keep // NT, (L - len_keep) // NT
    rest, mask, ids_keep, ids_mask = pl.pallas_call(
        functools.partial(_mega_kernel, L=L, E=E, len_keep=len_keep,
                          krows=krows, mrows=mrows),
        grid=(B, NT),
        in_specs=[pl.BlockSpec((1, R, C), lambda b, t: (b, 0, 0))],
        out_specs=[
            pl.BlockSpec((1, R, C), lambda b, t: (b, 0, 0)),
            pl.BlockSpec((1, R, C), lambda b, t: (b, 0, 0)),
            pl.BlockSpec((1, krows, E), lambda b, t: (b, t, 0)),
            pl.BlockSpec((1, mrows, E), lambda b, t: (b, t, 0)),
        ],
        out_shape=[
            jax.ShapeDtypeStruct((B, R, C), jnp.int32),
            jax.ShapeDtypeStruct((B, R, C), jnp.float32),
            jax.ShapeDtypeStruct((B, len_keep, E), jnp.int32),
            jax.ShapeDtypeStruct((B, L - len_keep, E), jnp.int32),
        ],
        scratch_shapes=[pltpu.VMEM((L, 1), jnp.int32)],
    )(shuf)

    return (mask.reshape(B, L), ids_keep, rest.reshape(B, L), ids_mask)


# speedup vs baseline: 2.5426x; 1.0121x over previous
import functools

import jax
import jax.numpy as jnp
from jax.experimental import pallas as pl
from jax.experimental.pallas import tpu as pltpu

MASK_RATIO_ = 0.75
R, C = 64, 128
KTILE = 2048


def _xor_shuffle(x, d):
    if d < C:
        bit = jax.lax.broadcasted_iota(jnp.int32, (R, C), 1) & d
        return jnp.where(bit != 0, pltpu.roll(x, d, 1), pltpu.roll(x, C - d, 1))
    s = d // C
    bit = jax.lax.broadcasted_iota(jnp.int32, (R, C), 0) & s
    return jnp.where(bit != 0, pltpu.roll(x, s, 0), pltpu.roll(x, R - s, 0))


def _sort_kernel(noise_ref, shuf_ref, *, L):
    ir = jax.lax.broadcasted_iota(jnp.int32, (R, C), 0)
    ic = jax.lax.broadcasted_iota(jnp.int32, (R, C), 1)
    idx = ir * C + ic
    m = (noise_ref[0, :, :] * float(1 << 23)).astype(jnp.int32)

    def bit_of(v):
        if v < C:
            return (ic & v) != 0
        return (ir & (v // C)) != 0

    key = m * C + ic
    k = 2
    while k <= C:
        d = k // 2
        while d >= 1:
            pk = _xor_shuffle(key, d)
            want_min = bit_of(k) == bit_of(d)
            take = (pk < key) == want_min
            key = jnp.where(take, pk, key)
            d //= 2
        k *= 2
    m = key >> 7
    idx = ir * C + (key & (C - 1))

    k = 2 * C
    while k <= L:
        d = k // 2
        while d >= 1:
            pm = _xor_shuffle(m, d)
            pidx = _xor_shuffle(idx, d)
            want_min = bit_of(k) == bit_of(d)
            take = (want_min & (pm < m)) | (~want_min & (m < pm))
            m = jnp.where(take, pm, m)
            idx = jnp.where(take, pidx, idx)
            d //= 2
        k *= 2

    ic1 = (ic & 1) != 0

    def nxt(x):
        a = pltpu.roll(x, C - 1, 1)
        return jnp.where(ic == C - 1, pltpu.roll(a, R - 1, 0), a)

    def prv(x):
        p = pltpu.roll(x, 1, 1)
        return jnp.where(ic == 0, pltpu.roll(p, 1, 0), p)

    mnext = nxt(m)
    mprev = prv(m)
    for parity in (0, 1, 0):
        inext = nxt(idx)
        iprev = prv(idx)
        is_low = ic1 if parity else jnp.logical_not(ic1)
        mp = jnp.where(is_low, mnext, mprev)
        idxp = jnp.where(is_low, inext, iprev)
        fixed = jnp.where(is_low, jnp.minimum(idx, idxp),
                          jnp.maximum(idx, idxp))
        idx = jnp.where(mp == m, fixed, idx)
    shuf_ref[0, :, :] = idx


def _mega_kernel(shg_ref, rest_ref, mask_ref, keep_ref, maskk_ref, colscr,
                 *, L, E, len_keep, krows, mrows):
    t = pl.program_id(1)

    @pl.when(t == 0)
    def _():
        sh = shg_ref[0, :, :]
        ident = (jax.lax.broadcasted_iota(jnp.int32, (C, C), 0) ==
                 jax.lax.broadcasted_iota(jnp.int32, (C, C), 1)
                 ).astype(jnp.float32)
        colmat = jax.lax.dot_general(
            ident, sh.astype(jnp.float32), (((1,), (1,)), ((), ())),
            precision=jax.lax.Precision.HIGHEST)
        colmat_i = colmat.astype(jnp.int32)
        for r in range(R):
            colscr[pl.ds(r * C, C), :] = colmat_i[:, r:r + 1]
        col = colscr[:, :]
        sh_row = jnp.concatenate(
            [sh[r:r + 1, :] for r in range(R)], axis=1)
        ihi = jax.lax.broadcasted_iota(jnp.int32, (R, 1), 0)
        ilo = jax.lax.broadcasted_iota(jnp.int32, (1, C), 1)
        a = ((sh_row >> 7) == ihi).astype(jnp.float32)
        kcol = jax.lax.broadcasted_iota(jnp.int32, (L, 1), 0)
        b = jnp.where((col & (C - 1)) == ilo, kcol, 0).astype(jnp.float32)
        mres = jnp.dot(a, b, precision=jax.lax.Precision.HIGHEST)
        rest_ref[0, :, :] = mres.astype(jnp.int32)
        mask_ref[0, :, :] = jnp.where(mres >= float(len_keep), 1.0, 0.0)

    kq = colscr[pl.ds(t * krows, krows), :]
    mq = colscr[pl.ds(len_keep + t * mrows, mrows), :]
    keep_ref[0, :, :] = jnp.broadcast_to(kq, (krows, E))
    maskk_ref[0, :, :] = jnp.broadcast_to(mq, (mrows, E))


def kernel(x):
    B, L, E = x.shape
    len_keep = int(L * (1.0 - MASK_RATIO_))
    noise = jax.random.uniform(
        jax.random.fold_in(jax.random.key(0), 1), (B, L), dtype=jnp.float32)
    noise_g = noise.reshape(B, R, C)

    shuf = pl.pallas_call(
        functools.partial(_sort_kernel, L=L),
        grid=(B,),
        in_specs=[pl.BlockSpec((1, R, C), lambda b: (b, 0, 0))],
        out_specs=pl.BlockSpec((1, R, C), lambda b: (b, 0, 0)),
        out_shape=jax.ShapeDtypeStruct((B, R, C), jnp.int32),
    )(noise_g)

    NT = 4
    krows, mrows = len_keep // NT, (L - len_keep) // NT
    rest, mask, ids_keep, ids_mask = pl.pallas_call(
        functools.partial(_mega_kernel, L=L, E=E, len_keep=len_keep,
                          krows=krows, mrows=mrows),
        grid=(B, NT),
        in_specs=[pl.BlockSpec((1, R, C), lambda b, t: (b, 0, 0))],
        out_specs=[
            pl.BlockSpec((1, R, C), lambda b, t: (b, 0, 0)),
            pl.BlockSpec((1, R, C), lambda b, t: (b, 0, 0)),
            pl.BlockSpec((1, krows, E), lambda b, t: (b, t, 0)),
            pl.BlockSpec((1, mrows, E), lambda b, t: (b, t, 0)),
        ],
        out_shape=[
            jax.ShapeDtypeStruct((B, R, C), jnp.int32),
            jax.ShapeDtypeStruct((B, R, C), jnp.float32),
            jax.ShapeDtypeStruct((B, len_keep, E), jnp.int32),
            jax.ShapeDtypeStruct((B, L - len_keep, E), jnp.int32),
        ],
        scratch_shapes=[pltpu.VMEM((L, 1), jnp.int32)],
    )(shuf)

    return (mask.reshape(B, L), ids_keep, rest.reshape(B, L), ids_mask)


# speedup vs baseline: 2.6048x; 1.0244x over previous
import functools

import jax
import jax.numpy as jnp
from jax.experimental import pallas as pl
from jax.experimental.pallas import tpu as pltpu

MASK_RATIO_ = 0.75
R, C = 64, 128
KTILE = 2048


def _xor_shuffle(x, d):
    if d < C:
        bit = jax.lax.broadcasted_iota(jnp.int32, (R, C), 1) & d
        return jnp.where(bit != 0, pltpu.roll(x, d, 1), pltpu.roll(x, C - d, 1))
    s = d // C
    bit = jax.lax.broadcasted_iota(jnp.int32, (R, C), 0) & s
    return jnp.where(bit != 0, pltpu.roll(x, s, 0), pltpu.roll(x, R - s, 0))


def _sort_body(noise, *, L):
    ir = jax.lax.broadcasted_iota(jnp.int32, (R, C), 0)
    ic = jax.lax.broadcasted_iota(jnp.int32, (R, C), 1)
    idx = ir * C + ic
    m = (noise * float(1 << 23)).astype(jnp.int32)

    def bit_of(v):
        if v < C:
            return (ic & v) != 0
        return (ir & (v // C)) != 0

    key = m * C + ic
    k = 2
    while k <= C:
        d = k // 2
        while d >= 1:
            pk = _xor_shuffle(key, d)
            want_min = bit_of(k) == bit_of(d)
            take = (pk < key) == want_min
            key = jnp.where(take, pk, key)
            d //= 2
        k *= 2
    m = key >> 7
    idx = ir * C + (key & (C - 1))

    k = 2 * C
    while k <= L:
        d = k // 2
        while d >= 1:
            pm = _xor_shuffle(m, d)
            pidx = _xor_shuffle(idx, d)
            want_min = bit_of(k) == bit_of(d)
            take = (want_min & (pm < m)) | (~want_min & (m < pm))
            m = jnp.where(take, pm, m)
            idx = jnp.where(take, pidx, idx)
            d //= 2
        k *= 2

    ic1 = (ic & 1) != 0

    def nxt(x):
        a = pltpu.roll(x, C - 1, 1)
        return jnp.where(ic == C - 1, pltpu.roll(a, R - 1, 0), a)

    def prv(x):
        p = pltpu.roll(x, 1, 1)
        return jnp.where(ic == 0, pltpu.roll(p, 1, 0), p)

    mnext = nxt(m)
    mprev = prv(m)
    for parity in (0, 1, 0):
        inext = nxt(idx)
        iprev = prv(idx)
        is_low = ic1 if parity else jnp.logical_not(ic1)
        mp = jnp.where(is_low, mnext, mprev)
        idxp = jnp.where(is_low, inext, iprev)
        fixed = jnp.where(is_low, jnp.minimum(idx, idxp),
                          jnp.maximum(idx, idxp))
        idx = jnp.where(mp == m, fixed, idx)
    return idx


def _mega_kernel(noise_ref, rest_ref, mask_ref, keep_ref, maskk_ref, colscr,
                 *, L, E, len_keep, krows, mrows):
    t = pl.program_id(1)

    @pl.when(t == 0)
    def _():
        sh = _sort_body(noise_ref[0, :, :], L=L)
        ident = (jax.lax.broadcasted_iota(jnp.int32, (C, C), 0) ==
                 jax.lax.broadcasted_iota(jnp.int32, (C, C), 1)
                 ).astype(jnp.float32)
        colmat = jax.lax.dot_general(
            ident, sh.astype(jnp.float32), (((1,), (1,)), ((), ())),
            precision=jax.lax.Precision.HIGHEST)
        colmat_i = colmat.astype(jnp.int32)
        for r in range(R):
            colscr[pl.ds(r * C, C), :] = colmat_i[:, r:r + 1]
        col = colscr[:, :]
        sh_row = jnp.concatenate(
            [sh[r:r + 1, :] for r in range(R)], axis=1)
        ihi = jax.lax.broadcasted_iota(jnp.int32, (R, 1), 0)
        ilo = jax.lax.broadcasted_iota(jnp.int32, (1, C), 1)
        a = ((sh_row >> 7) == ihi).astype(jnp.float32)
        kcol = jax.lax.broadcasted_iota(jnp.int32, (L, 1), 0)
        b = jnp.where((col & (C - 1)) == ilo, kcol, 0).astype(jnp.float32)
        mres = jnp.dot(a, b, precision=jax.lax.Precision.HIGHEST)
        rest_ref[0, :, :] = mres.astype(jnp.int32)
        mask_ref[0, :, :] = jnp.where(mres >= float(len_keep), 1.0, 0.0)

    kq = colscr[pl.ds(t * krows, krows), :]
    mq = colscr[pl.ds(len_keep + t * mrows, mrows), :]
    keep_ref[0, :, :] = jnp.broadcast_to(kq, (krows, E))
    maskk_ref[0, :, :] = jnp.broadcast_to(mq, (mrows, E))


def kernel(x):
    B, L, E = x.shape
    len_keep = int(L * (1.0 - MASK_RATIO_))
    noise = jax.random.uniform(
        jax.random.fold_in(jax.random.key(0), 1), (B, L), dtype=jnp.float32)
    noise_g = noise.reshape(B, R, C)

    NT = 4
    krows, mrows = len_keep // NT, (L - len_keep) // NT
    rest, mask, ids_keep, ids_mask = pl.pallas_call(
        functools.partial(_mega_kernel, L=L, E=E, len_keep=len_keep,
                          krows=krows, mrows=mrows),
        grid=(B, NT),
        in_specs=[pl.BlockSpec((1, R, C), lambda b, t: (b, 0, 0))],
        out_specs=[
            pl.BlockSpec((1, R, C), lambda b, t: (b, 0, 0)),
            pl.BlockSpec((1, R, C), lambda b, t: (b, 0, 0)),
            pl.BlockSpec((1, krows, E), lambda b, t: (b, t, 0)),
            pl.BlockSpec((1, mrows, E), lambda b, t: (b, t, 0)),
        ],
        out_shape=[
            jax.ShapeDtypeStruct((B, R, C), jnp.int32),
            jax.ShapeDtypeStruct((B, R, C), jnp.float32),
            jax.ShapeDtypeStruct((B, len_keep, E), jnp.int32),
            jax.ShapeDtypeStruct((B, L - len_keep, E), jnp.int32),
        ],
        scratch_shapes=[pltpu.VMEM((L, 1), jnp.int32)],
    )(noise_g)

    return (mask.reshape(B, L), ids_keep, rest.reshape(B, L), ids_mask)
